# baseline (speedup 1.0000x reference)
import functools

import jax
import jax.numpy as jnp
from jax import lax
from jax.experimental import pallas as pl
from jax.experimental.pallas import tpu as pltpu
from jax.experimental.pallas import tpu_sc as plsc

N_NODES = 10000
N_EDGES = 320000
D = 128
N_GRAPHS = 128

NC = 2
NS = 16
NW = NC * NS
CH = 128
E_PAD = 327680
CHUNKS = E_PAD // (NW * CH)
SEG_C0 = 158
SEG_C1 = 2 * CHUNKS - SEG_C0
CNT_C0 = 96
CNT_C1 = 2 * CHUNKS - CNT_C0
ACC_ROWS = 10240
ZROWS = ACC_ROWS // NS
CNT_ROWS = 10240
CZROWS = CNT_ROWS // NS

_HIGH = lax.Precision.HIGHEST


def _seg_body(x_hbm, src_hbm, dst_hbm, z2_hbm, s_out,
              acc, src_v, dst_v, rows_v, sems):
    cid = lax.axis_index("c")
    sid = lax.axis_index("s")
    base = lax.select(cid == 0, sid * SEG_C0, NS * SEG_C0 + sid * SEG_C1)
    nch = lax.select(cid == 0, SEG_C0, SEG_C1)

    pltpu.sync_copy(z2_hbm, acc.at[pl.ds(sid * ZROWS, ZROWS)])
    plsc.subcore_barrier()

    for b in range(2):
        pltpu.sync_copy(src_hbm.at[base + b], src_v[b])
        pltpu.sync_copy(dst_hbm.at[base + b], dst_v[b])
        pltpu.async_copy(x_hbm.at[src_v[b]], rows_v[b], sems[b])

    def round_(i, carry):
        for b in range(2):
            j = 2 * i + b
            pltpu.make_async_copy(x_hbm.at[src_v[b]], rows_v[b], sems[b]).wait()
            pltpu.sync_copy(rows_v[b], acc.at[dst_v[b]], add=True)
            jn = j + 2
            pltpu.sync_copy(src_hbm.at[base + jn], src_v[b])
            pltpu.sync_copy(dst_hbm.at[base + jn], dst_v[b])
            pltpu.async_copy(x_hbm.at[src_v[b]], rows_v[b], sems[b])
        return carry

    lax.fori_loop(0, nch // 2 - 1, round_, 0)
    for b in range(2):
        pltpu.make_async_copy(x_hbm.at[src_v[b]], rows_v[b], sems[b]).wait()
        pltpu.sync_copy(rows_v[b], acc.at[dst_v[b]], add=True)
    plsc.subcore_barrier()

    pltpu.sync_copy(acc.at[pl.ds(sid * ZROWS, ZROWS)],
                    s_out.at[pl.ds(cid * ACC_ROWS + sid * ZROWS, ZROWS)])


def _cnt_body(dst_hbm, z2_hbm, ones_hbm, c_out, acc, dst_v, rows_v, sems):
    cid = lax.axis_index("c")
    sid = lax.axis_index("s")
    base = lax.select(cid == 0, sid * CNT_C0, NS * CNT_C0 + sid * CNT_C1)
    nch = lax.select(cid == 0, CNT_C0, CNT_C1)
    pltpu.sync_copy(z2_hbm, acc.at[pl.ds(sid * ZROWS, ZROWS)])
    pltpu.sync_copy(ones_hbm, rows_v)
    plsc.subcore_barrier()

    for b in range(2):
        pltpu.sync_copy(dst_hbm.at[base + b], dst_v[b])
        pltpu.async_copy(rows_v, acc.at[dst_v[b]], sems[b], add=True)

    def round_(i, carry):
        for b in range(2):
            j = 2 * i + b
            pltpu.make_async_copy(rows_v, acc.at[dst_v[b]], sems[b]).wait()
            pltpu.sync_copy(dst_hbm.at[base + j + 2], dst_v[b])
            pltpu.async_copy(rows_v, acc.at[dst_v[b]], sems[b], add=True)
        return carry

    lax.fori_loop(0, nch // 2 - 1, round_, 0)
    for b in range(2):
        pltpu.make_async_copy(rows_v, acc.at[dst_v[b]], sems[b]).wait()
    plsc.subcore_barrier()
    pltpu.sync_copy(acc.at[pl.ds(sid * ZROWS, ZROWS)],
                    c_out.at[pl.ds(cid * ACC_ROWS + sid * ZROWS, ZROWS)])


def _sc_mesh():
    return plsc.VectorSubcoreMesh(core_axis_name="c", subcore_axis_name="s",
                                  num_cores=NC, num_subcores=NS)


@functools.lru_cache(maxsize=None)
def _make_seg_kernel():
    scratch = [
        pltpu.VMEM_SHARED((ACC_ROWS, D), jnp.float32),
        [pltpu.VMEM((CH,), jnp.int32)] * 2,
        [pltpu.VMEM((CH,), jnp.int32)] * 2,
        [pltpu.VMEM((CH, D), jnp.float32)] * 2,
        [pltpu.SemaphoreType.DMA] * 2,
    ]
    return pl.kernel(_seg_body,
                     out_type=jax.ShapeDtypeStruct((2 * ACC_ROWS, D), jnp.float32),
                     mesh=_sc_mesh(), scratch_types=scratch)


@functools.lru_cache(maxsize=None)
def _make_cnt_kernel():
    scratch = [
        pltpu.VMEM_SHARED((ACC_ROWS, D), jnp.float32),
        [pltpu.VMEM((CH,), jnp.int32)] * 2,
        pltpu.VMEM((CH, D), jnp.float32),
        [pltpu.SemaphoreType.DMA] * 2,
    ]
    return pl.kernel(_cnt_body,
                     out_type=jax.ShapeDtypeStruct((2 * ACC_ROWS, D), jnp.float32),
                     mesh=_sc_mesh(), scratch_types=scratch)


def _proj_body(x_ref, w_ref, b_ref, o_ref):
    o_ref[...] = jnp.dot(x_ref[...], w_ref[...],
                         preferred_element_type=jnp.float32,
                         precision=_HIGH) + b_ref[...]


def _tc_proj(x, w, b2):
    return pl.pallas_call(
        _proj_body,
        out_shape=jax.ShapeDtypeStruct((N_NODES, D), jnp.float32),
    )(x, w, b2)


def _conv_body(s0_ref, s1_ref, c0_ref, c1_ref, x_ref, wrel_ref, wroot_ref,
               b_ref, g_ref, be_ref, o_ref):
    s = s0_ref[...] + s1_ref[...]
    cnt = jnp.maximum(c0_ref[:, 0:1] + c1_ref[:, 0:1], 1.0)
    agg = jnp.dot(s, wrel_ref[...], preferred_element_type=jnp.float32,
                  precision=_HIGH) / cnt
    h = agg + jnp.dot(x_ref[...], wroot_ref[...],
                      preferred_element_type=jnp.float32,
                      precision=_HIGH) + b_ref[...]
    h = jnp.where(h > 0, h, jnp.exp(h) - 1.0)
    mu = jnp.mean(h, axis=-1, keepdims=True)
    var = jnp.mean((h - mu) ** 2, axis=-1, keepdims=True)
    o_ref[...] = (h - mu) / jnp.sqrt(var + 1e-5) * g_ref[...] + be_ref[...]


def _tc_conv(s0, s1, c0, c1, x, w_rel, w_root, b2, g2, be2):
    blk = 2000
    grid = N_NODES // blk
    return pl.pallas_call(
        _conv_body,
        grid=(grid,),
        in_specs=[
            pl.BlockSpec((blk, D), lambda i: (i, 0)),
            pl.BlockSpec((blk, D), lambda i: (i, 0)),
            pl.BlockSpec((blk, D), lambda i: (i, 0)),
            pl.BlockSpec((blk, D), lambda i: (i, 0)),
            pl.BlockSpec((blk, D), lambda i: (i, 0)),
            pl.BlockSpec((D, D), lambda i: (0, 0)),
            pl.BlockSpec((D, D), lambda i: (0, 0)),
            pl.BlockSpec((1, D), lambda i: (0, 0)),
            pl.BlockSpec((1, D), lambda i: (0, 0)),
            pl.BlockSpec((1, D), lambda i: (0, 0)),
        ],
        out_specs=pl.BlockSpec((blk, D), lambda i: (i, 0)),
        out_shape=jax.ShapeDtypeStruct((N_NODES, D), jnp.float32),
    )(s0, s1, c0, c1, x, w_rel, w_root, b2, g2, be2)


def _pool_body(x_ref, batch_ref, wh_ref, bh_ref, o_ref):
    b = batch_ref[...]
    gids = lax.broadcasted_iota(jnp.int32, (N_GRAPHS, N_NODES), 0)
    p = (gids == b).astype(jnp.float32)
    psum = jnp.dot(p, x_ref[...], preferred_element_type=jnp.float32,
                   precision=_HIGH)
    cnt_g = jnp.maximum(jnp.sum(p, axis=-1, keepdims=True), 1.0)
    pooled = psum / cnt_g
    o_ref[...] = jnp.dot(pooled, wh_ref[...], preferred_element_type=jnp.float32,
                         precision=_HIGH) + bh_ref[...]


def _tc_pool(x3, batch2, wh, bh2):
    return pl.pallas_call(
        _pool_body,
        out_shape=jax.ShapeDtypeStruct((N_GRAPHS, 2), jnp.float32),
    )(x3, batch2, wh, bh2)


def kernel(x_operator, edge_index_calledby, batch_operator, W_op, b_op,
           W_root, W_rel, b_conv, gamma, beta, W_mem, b_mem, W_time, b_time):
    src = edge_index_calledby[0].astype(jnp.int32)
    dst = edge_index_calledby[1].astype(jnp.int32)
    pad = E_PAD - N_EDGES
    src2 = jnp.concatenate([src, jnp.zeros((pad,), jnp.int32)]
                           ).reshape(NW * CHUNKS, CH)
    dst2 = jnp.concatenate([dst, jnp.full((pad,), N_NODES, jnp.int32)]
                           ).reshape(NW * CHUNKS, CH)
    batch2 = batch_operator.astype(jnp.int32).reshape(1, N_NODES)

    z2 = jnp.zeros((ZROWS, D), jnp.float32)
    on = jnp.ones((CH, D), jnp.float32)

    b2 = b_op.reshape(1, D)
    bc2 = b_conv.reshape(1, D)
    g2 = gamma.reshape(1, D)
    be2 = beta.reshape(1, D)
    wh = jnp.concatenate([W_mem, W_time], axis=1)
    bh2 = jnp.concatenate([b_mem, b_time]).reshape(1, 2)
    w_rel0 = W_rel[0]

    x1 = _tc_proj(x_operator, W_op, b2)
    cflat = _make_cnt_kernel()(dst2, z2, on)
    s1 = _make_seg_kernel()(x1, src2, dst2, z2)
    c0 = cflat[0:N_NODES]
    c1 = cflat[ACC_ROWS:ACC_ROWS + N_NODES]
    x2 = _tc_conv(s1[:N_NODES], s1[ACC_ROWS:ACC_ROWS + N_NODES],
                  c0, c1, x1, w_rel0, W_root, bc2, g2, be2)
    s2 = _make_seg_kernel()(x2, src2, dst2, z2)
    x3 = _tc_conv(s2[:N_NODES], s2[ACC_ROWS:ACC_ROWS + N_NODES],
                  c0, c1, x2, w_rel0, W_root, bc2, g2, be2)
    out = _tc_pool(x3, batch2, wh, bh2)
    return out[:, 0], out[:, 1]


# speedup vs baseline: 1.0565x; 1.0565x over previous
import functools

import jax
import jax.numpy as jnp
from jax import lax
from jax.experimental import pallas as pl
from jax.experimental.pallas import tpu as pltpu
from jax.experimental.pallas import tpu_sc as plsc

N_NODES = 10000
N_EDGES = 320000
D = 128
N_GRAPHS = 128

NC = 2
NS = 16
NW = NC * NS
CH = 128
E_PAD = 327680
CHUNKS = E_PAD // (NW * CH)
ACC_ROWS = 10240
ZROWS = ACC_ROWS // NS
SEG_C0 = 120
SEG_C1 = 2 * CHUNKS - SEG_C0
CNT_C0 = 96
CNT_C1 = 2 * CHUNKS - CNT_C0

_HIGH = lax.Precision.HIGHEST


def _seg_body(x_hbm, src_hbm, dst_hbm, z2_hbm, s_out,
              acc, src_v, dst_v, rows_v, gsem, isem):
    cid = lax.axis_index("c")
    sid = lax.axis_index("s")
    base = lax.select(cid == 0, sid * SEG_C0, NS * SEG_C0 + sid * SEG_C1)
    nch = lax.select(cid == 0, SEG_C0, SEG_C1)

    pltpu.sync_copy(z2_hbm, acc.at[pl.ds(sid * ZROWS, ZROWS)])
    plsc.subcore_barrier()

    def idx_fetch(b, j):
        pltpu.async_copy(src_hbm.at[base + j], src_v[b], isem[b])
        pltpu.async_copy(dst_hbm.at[base + j], dst_v[b], isem[b])

    def idx_wait(b, j):
        pltpu.make_async_copy(src_hbm.at[base + j], src_v[b], isem[b]).wait()
        pltpu.make_async_copy(dst_hbm.at[base + j], dst_v[b], isem[b]).wait()

    idx_fetch(0, 0)
    idx_fetch(1, 1)
    idx_wait(0, 0)
    pltpu.async_copy(x_hbm.at[src_v[0]], rows_v[0], gsem[0])
    idx_wait(1, 1)

    def round_(i, carry):
        j = 2 * i
        pltpu.async_copy(x_hbm.at[src_v[1]], rows_v[1], gsem[1])
        pltpu.make_async_copy(x_hbm.at[src_v[0]], rows_v[0], gsem[0]).wait()
        pltpu.sync_copy(rows_v[0], acc.at[dst_v[0]], add=True)
        idx_fetch(0, j + 2)
        idx_wait(0, j + 2)
        pltpu.async_copy(x_hbm.at[src_v[0]], rows_v[0], gsem[0])
        pltpu.make_async_copy(x_hbm.at[src_v[1]], rows_v[1], gsem[1]).wait()
        pltpu.sync_copy(rows_v[1], acc.at[dst_v[1]], add=True)
        idx_fetch(1, j + 3)
        idx_wait(1, j + 3)
        return carry

    lax.fori_loop(0, nch // 2 - 1, round_, 0)
    pltpu.async_copy(x_hbm.at[src_v[1]], rows_v[1], gsem[1])
    pltpu.make_async_copy(x_hbm.at[src_v[0]], rows_v[0], gsem[0]).wait()
    pltpu.sync_copy(rows_v[0], acc.at[dst_v[0]], add=True)
    pltpu.make_async_copy(x_hbm.at[src_v[1]], rows_v[1], gsem[1]).wait()
    pltpu.sync_copy(rows_v[1], acc.at[dst_v[1]], add=True)
    plsc.subcore_barrier()

    pltpu.sync_copy(acc.at[pl.ds(sid * ZROWS, ZROWS)],
                    s_out.at[pl.ds(cid * ACC_ROWS + sid * ZROWS, ZROWS)])


def _cnt_body(dst_hbm, z2_hbm, ones_hbm, c_out, acc, dst_v, rows_v, sems):
    cid = lax.axis_index("c")
    sid = lax.axis_index("s")
    base = lax.select(cid == 0, sid * CNT_C0, NS * CNT_C0 + sid * CNT_C1)
    nch = lax.select(cid == 0, CNT_C0, CNT_C1)
    pltpu.sync_copy(z2_hbm, acc.at[pl.ds(sid * ZROWS, ZROWS)])
    pltpu.sync_copy(ones_hbm, rows_v)
    plsc.subcore_barrier()

    for b in range(2):
        pltpu.sync_copy(dst_hbm.at[base + b], dst_v[b])
        pltpu.async_copy(rows_v, acc.at[dst_v[b]], sems[b], add=True)

    def round_(i, carry):
        for b in range(2):
            j = 2 * i + b
            pltpu.make_async_copy(rows_v, acc.at[dst_v[b]], sems[b]).wait()
            pltpu.sync_copy(dst_hbm.at[base + j + 2], dst_v[b])
            pltpu.async_copy(rows_v, acc.at[dst_v[b]], sems[b], add=True)
        return carry

    lax.fori_loop(0, nch // 2 - 1, round_, 0)
    for b in range(2):
        pltpu.make_async_copy(rows_v, acc.at[dst_v[b]], sems[b]).wait()
    plsc.subcore_barrier()
    pltpu.sync_copy(acc.at[pl.ds(sid * ZROWS, ZROWS)],
                    c_out.at[pl.ds(cid * ACC_ROWS + sid * ZROWS, ZROWS)])


def _sc_mesh():
    return plsc.VectorSubcoreMesh(core_axis_name="c", subcore_axis_name="s",
                                  num_cores=NC, num_subcores=NS)


@functools.lru_cache(maxsize=None)
def _make_seg_kernel():
    scratch = [
        pltpu.VMEM_SHARED((ACC_ROWS, D), jnp.float32),
        [pltpu.VMEM((CH,), jnp.int32)] * 2,
        [pltpu.VMEM((CH,), jnp.int32)] * 2,
        [pltpu.VMEM((CH, D), jnp.float32)] * 2,
        [pltpu.SemaphoreType.DMA] * 2,
        [pltpu.SemaphoreType.DMA] * 2,
    ]
    return pl.kernel(_seg_body,
                     out_type=jax.ShapeDtypeStruct((2 * ACC_ROWS, D), jnp.float32),
                     mesh=_sc_mesh(), scratch_types=scratch)


@functools.lru_cache(maxsize=None)
def _make_cnt_kernel():
    scratch = [
        pltpu.VMEM_SHARED((ACC_ROWS, D), jnp.float32),
        [pltpu.VMEM((CH,), jnp.int32)] * 2,
        pltpu.VMEM((CH, D), jnp.float32),
        [pltpu.SemaphoreType.DMA] * 2,
    ]
    return pl.kernel(_cnt_body,
                     out_type=jax.ShapeDtypeStruct((2 * ACC_ROWS, D), jnp.float32),
                     mesh=_sc_mesh(), scratch_types=scratch)


def _proj_body(x_ref, w_ref, b_ref, o_ref):
    o_ref[...] = jnp.dot(x_ref[...], w_ref[...],
                         preferred_element_type=jnp.float32,
                         precision=_HIGH) + b_ref[...]


def _tc_proj(x, w, b2):
    return pl.pallas_call(
        _proj_body,
        out_shape=jax.ShapeDtypeStruct((N_NODES, D), jnp.float32),
    )(x, w, b2)


def _conv_body(s0_ref, s1_ref, c0_ref, c1_ref, x_ref, wrel_ref, wroot_ref,
               b_ref, g_ref, be_ref, o_ref):
    s = s0_ref[...] + s1_ref[...]
    cnt = jnp.maximum(c0_ref[:, 0:1] + c1_ref[:, 0:1], 1.0)
    agg = jnp.dot(s, wrel_ref[...], preferred_element_type=jnp.float32,
                  precision=_HIGH) / cnt
    h = agg + jnp.dot(x_ref[...], wroot_ref[...],
                      preferred_element_type=jnp.float32,
                      precision=_HIGH) + b_ref[...]
    h = jnp.where(h > 0, h, jnp.exp(h) - 1.0)
    mu = jnp.mean(h, axis=-1, keepdims=True)
    var = jnp.mean((h - mu) ** 2, axis=-1, keepdims=True)
    o_ref[...] = (h - mu) / jnp.sqrt(var + 1e-5) * g_ref[...] + be_ref[...]


def _tc_conv(s0, s1, c0, c1, x, w_rel, w_root, b2, g2, be2):
    blk = 2000
    grid = N_NODES // blk
    return pl.pallas_call(
        _conv_body,
        grid=(grid,),
        in_specs=[
            pl.BlockSpec((blk, D), lambda i: (i, 0)),
            pl.BlockSpec((blk, D), lambda i: (i, 0)),
            pl.BlockSpec((blk, D), lambda i: (i, 0)),
            pl.BlockSpec((blk, D), lambda i: (i, 0)),
            pl.BlockSpec((blk, D), lambda i: (i, 0)),
            pl.BlockSpec((D, D), lambda i: (0, 0)),
            pl.BlockSpec((D, D), lambda i: (0, 0)),
            pl.BlockSpec((1, D), lambda i: (0, 0)),
            pl.BlockSpec((1, D), lambda i: (0, 0)),
            pl.BlockSpec((1, D), lambda i: (0, 0)),
        ],
        out_specs=pl.BlockSpec((blk, D), lambda i: (i, 0)),
        out_shape=jax.ShapeDtypeStruct((N_NODES, D), jnp.float32),
    )(s0, s1, c0, c1, x, w_rel, w_root, b2, g2, be2)


def _pool_body(x_ref, batch_ref, wh_ref, bh_ref, o_ref):
    b = batch_ref[...]
    gids = lax.broadcasted_iota(jnp.int32, (N_GRAPHS, N_NODES), 0)
    p = (gids == b).astype(jnp.float32)
    psum = jnp.dot(p, x_ref[...], preferred_element_type=jnp.float32,
                   precision=_HIGH)
    cnt_g = jnp.maximum(jnp.sum(p, axis=-1, keepdims=True), 1.0)
    pooled = psum / cnt_g
    o_ref[...] = jnp.dot(pooled, wh_ref[...], preferred_element_type=jnp.float32,
                         precision=_HIGH) + bh_ref[...]


def _tc_pool(x3, batch2, wh, bh2):
    return pl.pallas_call(
        _pool_body,
        out_shape=jax.ShapeDtypeStruct((N_GRAPHS, 2), jnp.float32),
    )(x3, batch2, wh, bh2)


def kernel(x_operator, edge_index_calledby, batch_operator, W_op, b_op,
           W_root, W_rel, b_conv, gamma, beta, W_mem, b_mem, W_time, b_time):
    src = edge_index_calledby[0].astype(jnp.int32)
    dst = edge_index_calledby[1].astype(jnp.int32)
    pad = E_PAD - N_EDGES
    src2 = jnp.concatenate([src, jnp.zeros((pad,), jnp.int32)]
                           ).reshape(NW * CHUNKS, CH)
    dst2 = jnp.concatenate([dst, jnp.full((pad,), N_NODES, jnp.int32)]
                           ).reshape(NW * CHUNKS, CH)
    batch2 = batch_operator.astype(jnp.int32).reshape(1, N_NODES)

    z2 = jnp.zeros((ZROWS, D), jnp.float32)
    on = jnp.ones((CH, D), jnp.float32)

    b2 = b_op.reshape(1, D)
    bc2 = b_conv.reshape(1, D)
    g2 = gamma.reshape(1, D)
    be2 = beta.reshape(1, D)
    wh = jnp.concatenate([W_mem, W_time], axis=1)
    bh2 = jnp.concatenate([b_mem, b_time]).reshape(1, 2)
    w_rel0 = W_rel[0]

    x1 = _tc_proj(x_operator, W_op, b2)
    cflat = _make_cnt_kernel()(dst2, z2, on)
    c0 = cflat[0:N_NODES]
    c1 = cflat[ACC_ROWS:ACC_ROWS + N_NODES]
    s1 = _make_seg_kernel()(x1, src2, dst2, z2)
    x2 = _tc_conv(s1[:N_NODES], s1[ACC_ROWS:ACC_ROWS + N_NODES],
                  c0, c1, x1, w_rel0, W_root, bc2, g2, be2)
    s2 = _make_seg_kernel()(x2, src2, dst2, z2)
    x3 = _tc_conv(s2[:N_NODES], s2[ACC_ROWS:ACC_ROWS + N_NODES],
                  c0, c1, x2, w_rel0, W_root, bc2, g2, be2)
    out = _tc_pool(x3, batch2, wh, bh2)
    return out[:, 0], out[:, 1]


# speedup vs baseline: 1.0635x; 1.0066x over previous
import functools

import jax
import jax.numpy as jnp
from jax import lax
from jax.experimental import pallas as pl
from jax.experimental.pallas import tpu as pltpu
from jax.experimental.pallas import tpu_sc as plsc

N_NODES = 10000
N_EDGES = 320000
D = 128
N_GRAPHS = 128

NC = 2
NS = 16
NW = NC * NS
CH = 128
E_PAD = 327680
CHUNKS = E_PAD // (NW * CH)
ACC_ROWS = 10240
ZROWS = ACC_ROWS // NS
SEG_C0 = 120
SEG_C1 = 2 * CHUNKS - SEG_C0
CNT_C0 = 84
CNT_C1 = 2 * CHUNKS - CNT_C0

_HIGH = lax.Precision.HIGHEST


def _seg_body(x_hbm, src_hbm, dst_hbm, z2_hbm, s_out,
              acc, src_v, dst_v, rows_v, gsem, isem):
    cid = lax.axis_index("c")
    sid = lax.axis_index("s")
    base = lax.select(cid == 0, sid * SEG_C0, NS * SEG_C0 + sid * SEG_C1)
    nch = lax.select(cid == 0, SEG_C0, SEG_C1)

    pltpu.sync_copy(z2_hbm, acc.at[pl.ds(sid * ZROWS, ZROWS)])
    plsc.subcore_barrier()

    def idx_fetch(b, j):
        pltpu.async_copy(src_hbm.at[base + j], src_v[b], isem[b])
        pltpu.async_copy(dst_hbm.at[base + j], dst_v[b], isem[b])

    def idx_wait(b, j):
        pltpu.make_async_copy(src_hbm.at[base + j], src_v[b], isem[b]).wait()
        pltpu.make_async_copy(dst_hbm.at[base + j], dst_v[b], isem[b]).wait()

    idx_fetch(0, 0)
    idx_fetch(1, 1)
    idx_wait(0, 0)
    pltpu.async_copy(x_hbm.at[src_v[0]], rows_v[0], gsem[0])
    idx_wait(1, 1)

    def round_(i, carry):
        j = 2 * i
        pltpu.async_copy(x_hbm.at[src_v[1]], rows_v[1], gsem[1])
        pltpu.make_async_copy(x_hbm.at[src_v[0]], rows_v[0], gsem[0]).wait()
        pltpu.async_copy(src_hbm.at[base + j + 2], src_v[0], isem[0])
        pltpu.sync_copy(rows_v[0], acc.at[dst_v[0]], add=True)
        pltpu.async_copy(dst_hbm.at[base + j + 2], dst_v[0], isem[0])
        idx_wait(0, j + 2)
        pltpu.async_copy(x_hbm.at[src_v[0]], rows_v[0], gsem[0])
        pltpu.make_async_copy(x_hbm.at[src_v[1]], rows_v[1], gsem[1]).wait()
        pltpu.async_copy(src_hbm.at[base + j + 3], src_v[1], isem[1])
        pltpu.sync_copy(rows_v[1], acc.at[dst_v[1]], add=True)
        pltpu.async_copy(dst_hbm.at[base + j + 3], dst_v[1], isem[1])
        idx_wait(1, j + 3)
        return carry

    lax.fori_loop(0, nch // 2 - 1, round_, 0)
    pltpu.async_copy(x_hbm.at[src_v[1]], rows_v[1], gsem[1])
    pltpu.make_async_copy(x_hbm.at[src_v[0]], rows_v[0], gsem[0]).wait()
    pltpu.sync_copy(rows_v[0], acc.at[dst_v[0]], add=True)
    pltpu.make_async_copy(x_hbm.at[src_v[1]], rows_v[1], gsem[1]).wait()
    pltpu.sync_copy(rows_v[1], acc.at[dst_v[1]], add=True)
    plsc.subcore_barrier()

    pltpu.sync_copy(acc.at[pl.ds(sid * ZROWS, ZROWS)],
                    s_out.at[pl.ds(cid * ACC_ROWS + sid * ZROWS, ZROWS)])


def _cnt_body(dst_hbm, z2_hbm, ones_hbm, c_out, acc, dst_v, rows_v, sems):
    cid = lax.axis_index("c")
    sid = lax.axis_index("s")
    base = lax.select(cid == 0, sid * CNT_C0, NS * CNT_C0 + sid * CNT_C1)
    nch = lax.select(cid == 0, CNT_C0, CNT_C1)
    pltpu.sync_copy(z2_hbm, acc.at[pl.ds(sid * ZROWS, ZROWS)])
    pltpu.sync_copy(ones_hbm, rows_v)
    plsc.subcore_barrier()

    for b in range(2):
        pltpu.sync_copy(dst_hbm.at[base + b], dst_v[b])
        pltpu.async_copy(rows_v, acc.at[dst_v[b]], sems[b], add=True)

    def round_(i, carry):
        for b in range(2):
            j = 2 * i + b
            pltpu.make_async_copy(rows_v, acc.at[dst_v[b]], sems[b]).wait()
            pltpu.sync_copy(dst_hbm.at[base + j + 2], dst_v[b])
            pltpu.async_copy(rows_v, acc.at[dst_v[b]], sems[b], add=True)
        return carry

    lax.fori_loop(0, nch // 2 - 1, round_, 0)
    for b in range(2):
        pltpu.make_async_copy(rows_v, acc.at[dst_v[b]], sems[b]).wait()
    plsc.subcore_barrier()
    pltpu.sync_copy(acc.at[pl.ds(sid * ZROWS, ZROWS)],
                    c_out.at[pl.ds(cid * ACC_ROWS + sid * ZROWS, ZROWS)])


def _sc_mesh():
    return plsc.VectorSubcoreMesh(core_axis_name="c", subcore_axis_name="s",
                                  num_cores=NC, num_subcores=NS)


@functools.lru_cache(maxsize=None)
def _make_seg_kernel():
    scratch = [
        pltpu.VMEM_SHARED((ACC_ROWS, D), jnp.float32),
        [pltpu.VMEM((CH,), jnp.int32)] * 2,
        [pltpu.VMEM((CH,), jnp.int32)] * 2,
        [pltpu.VMEM((CH, D), jnp.float32)] * 2,
        [pltpu.SemaphoreType.DMA] * 2,
        [pltpu.SemaphoreType.DMA] * 2,
    ]
    return pl.kernel(_seg_body,
                     out_type=jax.ShapeDtypeStruct((2 * ACC_ROWS, D), jnp.float32),
                     mesh=_sc_mesh(), scratch_types=scratch)


@functools.lru_cache(maxsize=None)
def _make_cnt_kernel():
    scratch = [
        pltpu.VMEM_SHARED((ACC_ROWS, D), jnp.float32),
        [pltpu.VMEM((CH,), jnp.int32)] * 2,
        pltpu.VMEM((CH, D), jnp.float32),
        [pltpu.SemaphoreType.DMA] * 2,
    ]
    return pl.kernel(_cnt_body,
                     out_type=jax.ShapeDtypeStruct((2 * ACC_ROWS, D), jnp.float32),
                     mesh=_sc_mesh(), scratch_types=scratch)


def _proj_body(x_ref, w_ref, b_ref, o_ref):
    o_ref[...] = jnp.dot(x_ref[...], w_ref[...],
                         preferred_element_type=jnp.float32,
                         precision=_HIGH) + b_ref[...]


def _tc_proj(x, w, b2):
    return pl.pallas_call(
        _proj_body,
        out_shape=jax.ShapeDtypeStruct((N_NODES, D), jnp.float32),
    )(x, w, b2)


def _conv_body(s0_ref, s1_ref, c0_ref, c1_ref, x_ref, wrel_ref, wroot_ref,
               b_ref, g_ref, be_ref, o_ref):
    s = s0_ref[...] + s1_ref[...]
    cnt = jnp.maximum(c0_ref[:, 0:1] + c1_ref[:, 0:1], 1.0)
    agg = jnp.dot(s, wrel_ref[...], preferred_element_type=jnp.float32,
                  precision=_HIGH) / cnt
    h = agg + jnp.dot(x_ref[...], wroot_ref[...],
                      preferred_element_type=jnp.float32,
                      precision=_HIGH) + b_ref[...]
    h = jnp.where(h > 0, h, jnp.exp(h) - 1.0)
    mu = jnp.mean(h, axis=-1, keepdims=True)
    var = jnp.mean((h - mu) ** 2, axis=-1, keepdims=True)
    o_ref[...] = (h - mu) / jnp.sqrt(var + 1e-5) * g_ref[...] + be_ref[...]


def _tc_conv(s0, s1, c0, c1, x, w_rel, w_root, b2, g2, be2):
    blk = 2000
    grid = N_NODES // blk
    return pl.pallas_call(
        _conv_body,
        grid=(grid,),
        in_specs=[
            pl.BlockSpec((blk, D), lambda i: (i, 0)),
            pl.BlockSpec((blk, D), lambda i: (i, 0)),
            pl.BlockSpec((blk, D), lambda i: (i, 0)),
            pl.BlockSpec((blk, D), lambda i: (i, 0)),
            pl.BlockSpec((blk, D), lambda i: (i, 0)),
            pl.BlockSpec((D, D), lambda i: (0, 0)),
            pl.BlockSpec((D, D), lambda i: (0, 0)),
            pl.BlockSpec((1, D), lambda i: (0, 0)),
            pl.BlockSpec((1, D), lambda i: (0, 0)),
            pl.BlockSpec((1, D), lambda i: (0, 0)),
        ],
        out_specs=pl.BlockSpec((blk, D), lambda i: (i, 0)),
        out_shape=jax.ShapeDtypeStruct((N_NODES, D), jnp.float32),
    )(s0, s1, c0, c1, x, w_rel, w_root, b2, g2, be2)


def _pool_body(x_ref, batch_ref, wh_ref, bh_ref, o_ref):
    b = batch_ref[...]
    gids = lax.broadcasted_iota(jnp.int32, (N_GRAPHS, N_NODES), 0)
    p = (gids == b).astype(jnp.float32)
    psum = jnp.dot(p, x_ref[...], preferred_element_type=jnp.float32,
                   precision=_HIGH)
    cnt_g = jnp.maximum(jnp.sum(p, axis=-1, keepdims=True), 1.0)
    pooled = psum / cnt_g
    o_ref[...] = jnp.dot(pooled, wh_ref[...], preferred_element_type=jnp.float32,
                         precision=_HIGH) + bh_ref[...]


def _tc_pool(x3, batch2, wh, bh2):
    return pl.pallas_call(
        _pool_body,
        out_shape=jax.ShapeDtypeStruct((N_GRAPHS, 2), jnp.float32),
    )(x3, batch2, wh, bh2)


def kernel(x_operator, edge_index_calledby, batch_operator, W_op, b_op,
           W_root, W_rel, b_conv, gamma, beta, W_mem, b_mem, W_time, b_time):
    src = edge_index_calledby[0].astype(jnp.int32)
    dst = edge_index_calledby[1].astype(jnp.int32)
    pad = E_PAD - N_EDGES
    src2 = jnp.concatenate([src, jnp.zeros((pad,), jnp.int32)]
                           ).reshape(NW * CHUNKS, CH)
    dst2 = jnp.concatenate([dst, jnp.full((pad,), N_NODES, jnp.int32)]
                           ).reshape(NW * CHUNKS, CH)
    batch2 = batch_operator.astype(jnp.int32).reshape(1, N_NODES)

    z2 = jnp.zeros((ZROWS, D), jnp.float32)
    on = jnp.ones((CH, D), jnp.float32)

    b2 = b_op.reshape(1, D)
    bc2 = b_conv.reshape(1, D)
    g2 = gamma.reshape(1, D)
    be2 = beta.reshape(1, D)
    wh = jnp.concatenate([W_mem, W_time], axis=1)
    bh2 = jnp.concatenate([b_mem, b_time]).reshape(1, 2)
    w_rel0 = W_rel[0]

    x1 = _tc_proj(x_operator, W_op, b2)
    cflat = _make_cnt_kernel()(dst2, z2, on)
    c0 = cflat[0:N_NODES]
    c1 = cflat[ACC_ROWS:ACC_ROWS + N_NODES]
    s1 = _make_seg_kernel()(x1, src2, dst2, z2)
    x2 = _tc_conv(s1[:N_NODES], s1[ACC_ROWS:ACC_ROWS + N_NODES],
                  c0, c1, x1, w_rel0, W_root, bc2, g2, be2)
    s2 = _make_seg_kernel()(x2, src2, dst2, z2)
    x3 = _tc_conv(s2[:N_NODES], s2[ACC_ROWS:ACC_ROWS + N_NODES],
                  c0, c1, x2, w_rel0, W_root, bc2, g2, be2)
    out = _tc_pool(x3, batch2, wh, bh2)
    return out[:, 0], out[:, 1]


# speedup vs baseline: 1.0936x; 1.0283x over previous
import functools

import jax
import jax.numpy as jnp
from jax import lax
from jax.experimental import pallas as pl
from jax.experimental.pallas import tpu as pltpu
from jax.experimental.pallas import tpu_sc as plsc

N_NODES = 10000
N_EDGES = 320000
D = 128
N_GRAPHS = 128

NC = 2
NS = 16
NW = NC * NS
CH = 128
E_PAD = 327680
CHUNKS = E_PAD // (NW * CH)
ACC_ROWS = 10240
ZROWS = ACC_ROWS // NS
SEG_C0 = 136
SEG_C1 = 2 * CHUNKS - SEG_C0
CNT_C0 = 84
CNT_C1 = 2 * CHUNKS - CNT_C0

_HIGH = lax.Precision.HIGHEST


def _seg_body(x_hbm, src_hbm, dst_hbm, z2_hbm, s_out,
              acc, src_v, dst_v, rows_v, gsem, isem):
    cid = lax.axis_index("c")
    sid = lax.axis_index("s")
    base = lax.select(cid == 0, sid * SEG_C0, NS * SEG_C0 + sid * SEG_C1)
    nch = lax.select(cid == 0, SEG_C0, SEG_C1)

    pltpu.sync_copy(z2_hbm, acc.at[pl.ds(sid * ZROWS, ZROWS)])
    plsc.subcore_barrier()

    def idx_fetch(b, j):
        pltpu.async_copy(src_hbm.at[base + j], src_v[b], isem[b])
        pltpu.async_copy(dst_hbm.at[base + j], dst_v[b], isem[b])

    def idx_wait(b, j):
        pltpu.make_async_copy(src_hbm.at[base + j], src_v[b], isem[b]).wait()
        pltpu.make_async_copy(dst_hbm.at[base + j], dst_v[b], isem[b]).wait()

    idx_fetch(0, 0)
    idx_fetch(1, 1)
    idx_wait(0, 0)
    pltpu.async_copy(x_hbm.at[src_v[0]], rows_v[0], gsem[0])
    idx_wait(1, 1)

    def round_(i, carry):
        j = 2 * i
        pltpu.async_copy(x_hbm.at[src_v[1]], rows_v[1], gsem[1])
        pltpu.make_async_copy(x_hbm.at[src_v[0]], rows_v[0], gsem[0]).wait()
        pltpu.async_copy(src_hbm.at[base + j + 2], src_v[0], isem[0])
        pltpu.sync_copy(rows_v[0], acc.at[dst_v[0]], add=True)
        pltpu.async_copy(dst_hbm.at[base + j + 2], dst_v[0], isem[0])
        idx_wait(0, j + 2)
        pltpu.async_copy(x_hbm.at[src_v[0]], rows_v[0], gsem[0])
        pltpu.make_async_copy(x_hbm.at[src_v[1]], rows_v[1], gsem[1]).wait()
        pltpu.async_copy(src_hbm.at[base + j + 3], src_v[1], isem[1])
        pltpu.sync_copy(rows_v[1], acc.at[dst_v[1]], add=True)
        pltpu.async_copy(dst_hbm.at[base + j + 3], dst_v[1], isem[1])
        idx_wait(1, j + 3)
        return carry

    lax.fori_loop(0, nch // 2 - 1, round_, 0)
    pltpu.async_copy(x_hbm.at[src_v[1]], rows_v[1], gsem[1])
    pltpu.make_async_copy(x_hbm.at[src_v[0]], rows_v[0], gsem[0]).wait()
    pltpu.sync_copy(rows_v[0], acc.at[dst_v[0]], add=True)
    pltpu.make_async_copy(x_hbm.at[src_v[1]], rows_v[1], gsem[1]).wait()
    pltpu.sync_copy(rows_v[1], acc.at[dst_v[1]], add=True)
    plsc.subcore_barrier()

    pltpu.sync_copy(acc.at[pl.ds(sid * ZROWS, ZROWS)],
                    s_out.at[pl.ds(cid * ACC_ROWS + sid * ZROWS, ZROWS)])


def _cnt_body(dst_hbm, z2_hbm, ones_hbm, c_out, acc, dst_v, rows_v, sems):
    cid = lax.axis_index("c")
    sid = lax.axis_index("s")
    base = lax.select(cid == 0, sid * CNT_C0, NS * CNT_C0 + sid * CNT_C1)
    nch = lax.select(cid == 0, CNT_C0, CNT_C1)
    pltpu.sync_copy(z2_hbm, acc.at[pl.ds(sid * ZROWS, ZROWS)])
    pltpu.sync_copy(ones_hbm, rows_v)
    plsc.subcore_barrier()

    for b in range(2):
        pltpu.sync_copy(dst_hbm.at[base + b], dst_v[b])
        pltpu.async_copy(rows_v, acc.at[dst_v[b]], sems[b], add=True)

    def round_(i, carry):
        for b in range(2):
            j = 2 * i + b
            pltpu.make_async_copy(rows_v, acc.at[dst_v[b]], sems[b]).wait()
            pltpu.sync_copy(dst_hbm.at[base + j + 2], dst_v[b])
            pltpu.async_copy(rows_v, acc.at[dst_v[b]], sems[b], add=True)
        return carry

    lax.fori_loop(0, nch // 2 - 1, round_, 0)
    for b in range(2):
        pltpu.make_async_copy(rows_v, acc.at[dst_v[b]], sems[b]).wait()
    plsc.subcore_barrier()
    pltpu.sync_copy(acc.at[pl.ds(sid * ZROWS, ZROWS)],
                    c_out.at[pl.ds(cid * ACC_ROWS + sid * ZROWS, ZROWS)])


def _sc_mesh():
    return plsc.VectorSubcoreMesh(core_axis_name="c", subcore_axis_name="s",
                                  num_cores=NC, num_subcores=NS)


@functools.lru_cache(maxsize=None)
def _make_seg_kernel():
    scratch = [
        pltpu.VMEM_SHARED((ACC_ROWS, D), jnp.float32),
        [pltpu.VMEM((CH,), jnp.int32)] * 2,
        [pltpu.VMEM((CH,), jnp.int32)] * 2,
        [pltpu.VMEM((CH, D), jnp.float32)] * 2,
        [pltpu.SemaphoreType.DMA] * 2,
        [pltpu.SemaphoreType.DMA] * 2,
    ]
    return pl.kernel(_seg_body,
                     out_type=jax.ShapeDtypeStruct((2 * ACC_ROWS, D), jnp.float32),
                     mesh=_sc_mesh(), scratch_types=scratch)


@functools.lru_cache(maxsize=None)
def _make_cnt_kernel():
    scratch = [
        pltpu.VMEM_SHARED((ACC_ROWS, D), jnp.float32),
        [pltpu.VMEM((CH,), jnp.int32)] * 2,
        pltpu.VMEM((CH, D), jnp.float32),
        [pltpu.SemaphoreType.DMA] * 2,
    ]
    return pl.kernel(_cnt_body,
                     out_type=jax.ShapeDtypeStruct((2 * ACC_ROWS, D), jnp.float32),
                     mesh=_sc_mesh(), scratch_types=scratch)


def _proj_body(x_ref, w_ref, b_ref, o_ref):
    o_ref[...] = jnp.dot(x_ref[...], w_ref[...],
                         preferred_element_type=jnp.float32,
                         precision=_HIGH) + b_ref[...]


def _tc_proj(x, w, b2):
    return pl.pallas_call(
        _proj_body,
        out_shape=jax.ShapeDtypeStruct((N_NODES, D), jnp.float32),
    )(x, w, b2)


def _conv_body(s0_ref, s1_ref, c0_ref, c1_ref, x_ref, wrel_ref, wroot_ref,
               b_ref, g_ref, be_ref, o_ref):
    s = s0_ref[...] + s1_ref[...]
    cnt = jnp.maximum(c0_ref[:, 0:1] + c1_ref[:, 0:1], 1.0)
    agg = jnp.dot(s, wrel_ref[...], preferred_element_type=jnp.float32,
                  precision=_HIGH) / cnt
    h = agg + jnp.dot(x_ref[...], wroot_ref[...],
                      preferred_element_type=jnp.float32,
                      precision=_HIGH) + b_ref[...]
    h = jnp.where(h > 0, h, jnp.exp(h) - 1.0)
    mu = jnp.mean(h, axis=-1, keepdims=True)
    var = jnp.mean((h - mu) ** 2, axis=-1, keepdims=True)
    o_ref[...] = (h - mu) / jnp.sqrt(var + 1e-5) * g_ref[...] + be_ref[...]


def _tc_conv(s0, s1, c0, c1, x, w_rel, w_root, b2, g2, be2):
    blk = 2000
    grid = N_NODES // blk
    return pl.pallas_call(
        _conv_body,
        grid=(grid,),
        in_specs=[
            pl.BlockSpec((blk, D), lambda i: (i, 0)),
            pl.BlockSpec((blk, D), lambda i: (i, 0)),
            pl.BlockSpec((blk, D), lambda i: (i, 0)),
            pl.BlockSpec((blk, D), lambda i: (i, 0)),
            pl.BlockSpec((blk, D), lambda i: (i, 0)),
            pl.BlockSpec((D, D), lambda i: (0, 0)),
            pl.BlockSpec((D, D), lambda i: (0, 0)),
            pl.BlockSpec((1, D), lambda i: (0, 0)),
            pl.BlockSpec((1, D), lambda i: (0, 0)),
            pl.BlockSpec((1, D), lambda i: (0, 0)),
        ],
        out_specs=pl.BlockSpec((blk, D), lambda i: (i, 0)),
        out_shape=jax.ShapeDtypeStruct((N_NODES, D), jnp.float32),
    )(s0, s1, c0, c1, x, w_rel, w_root, b2, g2, be2)


def _pool_body(x_ref, batch_ref, wh_ref, bh_ref, o_ref):
    b = batch_ref[...]
    gids = lax.broadcasted_iota(jnp.int32, (N_GRAPHS, N_NODES), 0)
    p = (gids == b).astype(jnp.float32)
    psum = jnp.dot(p, x_ref[...], preferred_element_type=jnp.float32,
                   precision=_HIGH)
    cnt_g = jnp.maximum(jnp.sum(p, axis=-1, keepdims=True), 1.0)
    pooled = psum / cnt_g
    o_ref[...] = jnp.dot(pooled, wh_ref[...], preferred_element_type=jnp.float32,
                         precision=_HIGH) + bh_ref[...]


def _tc_pool(x3, batch2, wh, bh2):
    return pl.pallas_call(
        _pool_body,
        out_shape=jax.ShapeDtypeStruct((N_GRAPHS, 2), jnp.float32),
    )(x3, batch2, wh, bh2)


def kernel(x_operator, edge_index_calledby, batch_operator, W_op, b_op,
           W_root, W_rel, b_conv, gamma, beta, W_mem, b_mem, W_time, b_time):
    src = edge_index_calledby[0].astype(jnp.int32)
    dst = edge_index_calledby[1].astype(jnp.int32)
    pad = E_PAD - N_EDGES
    src2 = jnp.concatenate([src, jnp.zeros((pad,), jnp.int32)]
                           ).reshape(NW * CHUNKS, CH)
    dst2 = jnp.concatenate([dst, jnp.full((pad,), N_NODES, jnp.int32)]
                           ).reshape(NW * CHUNKS, CH)
    batch2 = batch_operator.astype(jnp.int32).reshape(1, N_NODES)

    z2 = jnp.zeros((ZROWS, D), jnp.float32)
    on = jnp.ones((CH, D), jnp.float32)

    b2 = b_op.reshape(1, D)
    bc2 = b_conv.reshape(1, D)
    g2 = gamma.reshape(1, D)
    be2 = beta.reshape(1, D)
    wh = jnp.concatenate([W_mem, W_time], axis=1)
    bh2 = jnp.concatenate([b_mem, b_time]).reshape(1, 2)
    w_rel0 = W_rel[0]

    x1 = _tc_proj(x_operator, W_op, b2)
    cflat = _make_cnt_kernel()(dst2, z2, on)
    c0 = cflat[0:N_NODES]
    c1 = cflat[ACC_ROWS:ACC_ROWS + N_NODES]
    s1 = _make_seg_kernel()(x1, src2, dst2, z2)
    x2 = _tc_conv(s1[:N_NODES], s1[ACC_ROWS:ACC_ROWS + N_NODES],
                  c0, c1, x1, w_rel0, W_root, bc2, g2, be2)
    s2 = _make_seg_kernel()(x2, src2, dst2, z2)
    x3 = _tc_conv(s2[:N_NODES], s2[ACC_ROWS:ACC_ROWS + N_NODES],
                  c0, c1, x2, w_rel0, W_root, bc2, g2, be2)
    out = _tc_pool(x3, batch2, wh, bh2)
    return out[:, 0], out[:, 1]
